# baseline (speedup 1.0000x reference)
import functools

import jax
import jax.numpy as jnp
from jax import lax
from jax.experimental import pallas as pl
from jax.experimental.pallas import tpu as pltpu
from jax.experimental.pallas import tpu_sc as plsc

VOCAB = 1000000
DIM = 64
B = 16384
WIN = 20
NNEG = 20

NC = 2
NS = 16
LANES = 16
NW = NC * NS
CPW = B // NW
S = 32
NCHUNK = CPW // S
S20 = S * WIN
IDX_ROWS = S20 // 128
NJ = DIM // LANES

_mesh = plsc.VectorSubcoreMesh(core_axis_name="c", subcore_axis_name="s")


def _tree_sum(gs):
    while len(gs) > 1:
        nxt = [gs[i] + gs[i + 1] for i in range(0, len(gs) - 1, 2)]
        if len(gs) % 2:
            nxt.append(gs[-1])
        gs = nxt
    return gs[0]


@functools.partial(
    pl.kernel,
    out_type=[
        jax.ShapeDtypeStruct((B,), jnp.float32),
        jax.ShapeDtypeStruct((B * NNEG,), jnp.float32),
    ],
    name="cbow_scores",
    mesh=_mesh,
    compiler_params=pltpu.CompilerParams(
        needs_layout_passes=False, use_tc_tiling_on_sc=False),
    scratch_types=[
        pltpu.VMEM((CPW,), jnp.int32),
        pltpu.VMEM((CPW * WIN // 128, 128), jnp.int32),
        pltpu.VMEM((CPW * NNEG // 128, 128), jnp.int32),
        pltpu.VMEM((S20, DIM), jnp.float32),
        pltpu.VMEM((S20, DIM), jnp.float32),
        pltpu.VMEM((S, DIM), jnp.float32),
        pltpu.VMEM((S,), jnp.float32),
        pltpu.VMEM((S20,), jnp.float32),
        pltpu.SemaphoreType.DMA,
    ],
)
def _sc_scores(ctx_hbm, tgt_hbm, neg_hbm, w_hbm,
               pos_hbm, nout_hbm,
               cidx_v, tidx_v, nidx_v, trows_v, nrows_v, crows_v,
               pos_v, nsc_v, sem):
    wid = lax.axis_index("s") * NC + lax.axis_index("c")
    wrows = CPW * WIN // 128

    pltpu.sync_copy(ctx_hbm.at[pl.ds(wid * CPW, CPW)], cidx_v)
    pltpu.sync_copy(tgt_hbm.at[pl.ds(wid * wrows, wrows)], tidx_v)
    pltpu.sync_copy(neg_hbm.at[pl.ds(wid * wrows, wrows)], nidx_v)

    lane_iota = lax.iota(jnp.int32, LANES)

    def chunk_body(c, carry):
        base = wid * CPW + c * S

        copies = []
        for j in range(IDX_ROWS):
            copies.append(pltpu.async_copy(
                w_hbm.at[tidx_v.at[c * IDX_ROWS + j]],
                trows_v.at[pl.ds(j * 128, 128)], sem))
            copies.append(pltpu.async_copy(
                w_hbm.at[nidx_v.at[c * IDX_ROWS + j]],
                nrows_v.at[pl.ds(j * 128, 128)], sem))
        copies.append(pltpu.async_copy(
            w_hbm.at[cidx_v.at[pl.ds(c * S, S)]], crows_v, sem))
        for cp in copies:
            cp.wait()

        for b0 in range(0, S, LANES):
            zero = jnp.zeros((LANES,), jnp.float32)

            def ebody(il, scores, b0=b0):
                r0e = (b0 + il) * WIN
                t = [trows_v[r0e, pl.ds(LANES * j, LANES)] for j in range(NJ)]
                for w in range(1, WIN):
                    rw = [trows_v[r0e + w, pl.ds(LANES * j, LANES)]
                          for j in range(NJ)]
                    t = [t[j] + rw[j] for j in range(NJ)]
                m = lane_iota == il
                pv = _tree_sum([t[j] * crows_v[b0 + il, pl.ds(LANES * j, LANES)]
                                for j in range(NJ)])
                out = [jnp.where(m, jnp.full((LANES,), jnp.sum(pv)), scores[0])]
                for n in range(NNEG):
                    nv = _tree_sum([
                        t[j] * nrows_v[r0e + n, pl.ds(LANES * j, LANES)]
                        for j in range(NJ)])
                    out.append(jnp.where(
                        m, jnp.full((LANES,), jnp.sum(nv)), scores[1 + n]))
                return tuple(out)

            res = lax.fori_loop(0, LANES, ebody, (zero,) * (1 + NNEG))
            pos_v[pl.ds(b0, LANES)] = res[0]
            for n in range(NNEG):
                nsc_v[pl.ds(n * S + b0, LANES)] = res[1 + n]

        pltpu.sync_copy(pos_v, pos_hbm.at[pl.ds(base, S)])
        pltpu.sync_copy(nsc_v, nout_hbm.at[pl.ds((wid * NCHUNK + c) * S20, S20)])
        return carry

    lax.fori_loop(0, NCHUNK, chunk_body, 0)


def _loss_body(pos_ref, neg_ref, out_ref):
    p = jnp.clip(pos_ref[...] * (1.0 / WIN), -10.0, 10.0)
    n = jnp.clip(neg_ref[...] * (1.0 / WIN), -10.0, 10.0)
    lp = jnp.sum(jnp.log1p(jnp.exp(-p)))
    ln = jnp.sum(jnp.log1p(jnp.exp(n)))
    out_ref[...] = ((lp + ln) * (1.0 / B)).reshape(1, 1)


_loss_tc = pl.pallas_call(
    _loss_body,
    out_shape=jax.ShapeDtypeStruct((1, 1), jnp.float32),
)


@jax.jit
def kernel(context, target, negatives, W_target, W_context):
    w_all = jnp.concatenate([W_target, W_context], axis=0)
    tgt2d = target.reshape(-1, 128)
    neg2d = (negatives + VOCAB).reshape(-1, 128)
    pos_raw, neg_raw = _sc_scores(context + VOCAB, tgt2d, neg2d, w_all)
    out = _loss_tc(pos_raw.reshape(128, 128), neg_raw.reshape(-1, 128))
    return out[0, 0]


# speedup vs baseline: 1.5961x; 1.5961x over previous
import functools

import jax
import jax.numpy as jnp
from jax import lax
from jax.experimental import pallas as pl
from jax.experimental.pallas import tpu as pltpu
from jax.experimental.pallas import tpu_sc as plsc

VOCAB = 1000000
DIM = 64
B = 16384
WIN = 20
NNEG = 20

NC = 2
NS = 16
LANES = 16
NW = NC * NS
CPW = B // NW
S = 32
NCHUNK = CPW // S
S20 = S * WIN
IDX_ROWS = S20 // 128
NJ = DIM // LANES

_mesh = plsc.VectorSubcoreMesh(core_axis_name="c", subcore_axis_name="s")


def _tree_sum(gs):
    while len(gs) > 1:
        nxt = [gs[i] + gs[i + 1] for i in range(0, len(gs) - 1, 2)]
        if len(gs) % 2:
            nxt.append(gs[-1])
        gs = nxt
    return gs[0]


@functools.partial(
    pl.kernel,
    out_type=[
        jax.ShapeDtypeStruct((B,), jnp.float32),
        jax.ShapeDtypeStruct((B * NNEG,), jnp.float32),
    ],
    name="cbow_scores",
    mesh=_mesh,
    compiler_params=pltpu.CompilerParams(
        needs_layout_passes=False, use_tc_tiling_on_sc=False),
    scratch_types=[
        pltpu.VMEM((CPW,), jnp.int32),
        pltpu.VMEM((WIN, CPW), jnp.int32),
        pltpu.VMEM((NNEG, CPW), jnp.int32),
        pltpu.VMEM((S20,), jnp.int32),
        pltpu.VMEM((S20,), jnp.int32),
        pltpu.VMEM((S20, DIM), jnp.float32),
        pltpu.VMEM((S20, DIM), jnp.float32),
        pltpu.VMEM((S, DIM), jnp.float32),
        pltpu.VMEM((S,), jnp.float32),
        pltpu.VMEM((S20,), jnp.float32),
        pltpu.SemaphoreType.DMA,
    ],
)
def _sc_scores(ctx_hbm, tgt_hbm, neg_hbm, wt_hbm, wc_hbm,
               pos_hbm, nout_hbm,
               cidx_v, tidx_v, nidx_v, tci_v, nci_v, trows_v, nrows_v, crows_v,
               pos_v, nsc_v, sem):
    wid = lax.axis_index("s") * NC + lax.axis_index("c")

    pltpu.sync_copy(ctx_hbm.at[pl.ds(wid * CPW, CPW)], cidx_v)
    pltpu.sync_copy(
        tgt_hbm.at[pl.ds(0, WIN), pl.ds(wid * CPW, CPW)], tidx_v)
    pltpu.sync_copy(
        neg_hbm.at[pl.ds(0, NNEG), pl.ds(wid * CPW, CPW)], nidx_v)

    lane_iota = lax.iota(jnp.int32, LANES)

    def chunk_body(c, carry):
        base = wid * CPW + c * S

        for w in range(WIN):
            for h in range(0, S, LANES):
                tci_v[pl.ds(w * S + h, LANES)] = (
                    tidx_v[w, pl.ds(c * S + h, LANES)])
                nci_v[pl.ds(w * S + h, LANES)] = (
                    nidx_v[w, pl.ds(c * S + h, LANES)])

        copies = []
        for j in range(IDX_ROWS):
            copies.append(pltpu.async_copy(
                wt_hbm.at[tci_v.at[pl.ds(j * 128, 128)]],
                trows_v.at[pl.ds(j * 128, 128)], sem))
            copies.append(pltpu.async_copy(
                wc_hbm.at[nci_v.at[pl.ds(j * 128, 128)]],
                nrows_v.at[pl.ds(j * 128, 128)], sem))
        copies.append(pltpu.async_copy(
            wc_hbm.at[cidx_v.at[pl.ds(c * S, S)]], crows_v, sem))
        for cp in copies:
            cp.wait()

        for b0 in range(0, S, LANES):
            zero = jnp.zeros((LANES,), jnp.float32)

            def ebody(il, scores, b0=b0):
                e = b0 + il
                t = [trows_v[e, pl.ds(LANES * j, LANES)] for j in range(NJ)]
                for w in range(1, WIN):
                    rw = [trows_v[w * S + e, pl.ds(LANES * j, LANES)]
                          for j in range(NJ)]
                    t = [t[j] + rw[j] for j in range(NJ)]
                m = lane_iota == il
                pv = _tree_sum([t[j] * crows_v[e, pl.ds(LANES * j, LANES)]
                                for j in range(NJ)])
                out = [jnp.where(m, jnp.full((LANES,), jnp.sum(pv)), scores[0])]
                for n in range(NNEG):
                    nv = _tree_sum([
                        t[j] * nrows_v[n * S + e, pl.ds(LANES * j, LANES)]
                        for j in range(NJ)])
                    out.append(jnp.where(
                        m, jnp.full((LANES,), jnp.sum(nv)), scores[1 + n]))
                return tuple(out)

            res = lax.fori_loop(0, LANES, ebody, (zero,) * (1 + NNEG))
            pos_v[pl.ds(b0, LANES)] = res[0]
            for n in range(NNEG):
                nsc_v[pl.ds(n * S + b0, LANES)] = res[1 + n]

        pltpu.sync_copy(pos_v, pos_hbm.at[pl.ds(base, S)])
        pltpu.sync_copy(nsc_v, nout_hbm.at[pl.ds((wid * NCHUNK + c) * S20, S20)])
        return carry

    lax.fori_loop(0, NCHUNK, chunk_body, 0)


def _loss_body(pos_ref, neg_ref, out_ref):
    p = jnp.clip(pos_ref[...] * (1.0 / WIN), -10.0, 10.0)
    n = jnp.clip(neg_ref[...] * (1.0 / WIN), -10.0, 10.0)
    lp = jnp.sum(jnp.log1p(jnp.exp(-p)))
    ln = jnp.sum(jnp.log1p(jnp.exp(n)))
    out_ref[...] = ((lp + ln) * (1.0 / B)).reshape(1, 1)


_loss_tc = pl.pallas_call(
    _loss_body,
    out_shape=jax.ShapeDtypeStruct((1, 1), jnp.float32),
)


@jax.jit
def kernel(context, target, negatives, W_target, W_context):
    pos_raw, neg_raw = _sc_scores(
        context, target.T, negatives.T, W_target, W_context)
    out = _loss_tc(pos_raw.reshape(128, 128), neg_raw.reshape(-1, 128))
    return out[0, 0]
